# baseline (speedup 1.0000x reference)
import functools

import jax
import jax.numpy as jnp
from jax.experimental import pallas as pl
from jax.experimental.pallas import tpu as pltpu

B = 64
K = 4
V = 100000
PLACEHOLDER = -1

BB = 8
VB = 8192
NC = (V + VB - 1) // VB


def _rs_kernel(d_tok_ref, u_ref, bonus_ref, greedy_ref,
               t_ref, d_ref, q_ref, out_ref,
               run_rv, run_ri, run_tv, run_ti, acc_pd, acc_pt):
    c = pl.program_id(1)

    @pl.when(c == 0)
    def _init():
        run_rv[...] = jnp.full((BB, K), -1, jnp.int32)
        run_ri[...] = jnp.zeros((BB, K), jnp.int32)
        run_tv[...] = jnp.full((BB, K), -1, jnp.int32)
        run_ti[...] = jnp.zeros((BB, K), jnp.int32)
        acc_pd[...] = jnp.zeros((BB, K), jnp.float32)
        acc_pt[...] = jnp.zeros((BB, K), jnp.float32)

    t = t_ref[...]
    d = d_ref[...]
    qv = q_ref[...]

    lane = jax.lax.broadcasted_iota(jnp.int32, (1, 1, VB), 2)
    gid = lane + c * VB
    valid = gid < V

    ratio = jnp.maximum(t - d, 0.0) / qv
    rbits = jnp.where(ratio != ratio, jnp.int32(0x7FFFFFFF),
                      jax.lax.bitcast_convert_type(ratio, jnp.int32))
    rkey = jnp.where(valid, rbits, -1)
    tkey = jnp.where(valid, jax.lax.bitcast_convert_type(t, jnp.int32), -1)

    def upd_argmax(x, rv_ref, ri_ref):
        bval = jnp.max(x, axis=-1)
        eq = x == bval[..., None]
        bidx = jnp.min(jnp.where(eq, gid, V), axis=-1)
        better = bval > rv_ref[...]
        ri_ref[...] = jnp.where(better, bidx, ri_ref[...])
        rv_ref[...] = jnp.maximum(rv_ref[...], bval)

    upd_argmax(rkey, run_rv, run_ri)
    upd_argmax(tkey, run_tv, run_ti)

    tok = d_tok_ref[...]
    hit = gid == tok[..., None]
    acc_pt[...] += jnp.sum(jnp.where(hit, t, 0.0), axis=-1)
    acc_pd[...] += jnp.sum(jnp.where(hit, d, 0.0), axis=-1)

    @pl.when(c == NC - 1)
    def _final():
        pd = acc_pd[...]
        pt = acc_pt[...]
        u = u_ref[...]
        tokf = d_tok_ref[...]
        bonus = bonus_ref[...]
        greedy = greedy_ref[...] != 0

        r = jnp.where(pd > 0, pt / jnp.where(pd > 0, pd, 1.0), 0.0)
        accept = ((pd > 0) & (r >= u)).astype(jnp.int32)
        c1 = accept[:, 0:1]
        c2 = c1 * accept[:, 1:2]
        c3 = c2 * accept[:, 2:3]
        c4 = c3 * accept[:, 3:4]
        num_acc = c1 + c2 + c3 + c4

        pos = jax.lax.broadcasted_iota(jnp.int32, (BB, K), 1)
        rand_tok = jnp.where(pos < num_acc, tokf,
                             jnp.where(pos == num_acc, run_ri[...], PLACEHOLDER))
        rand_bonus = jnp.where(num_acc == K, bonus, PLACEHOLDER)

        t_arg = run_ti[...]
        match = (tokf == t_arg).astype(jnp.int32)
        m1 = match[:, 0:1]
        m2 = m1 * match[:, 1:2]
        m3 = m2 * match[:, 2:3]
        m4 = m3 * match[:, 3:4]
        num_match = m1 + m2 + m3 + m4
        greedy_tok = jnp.where(pos <= num_match, t_arg, PLACEHOLDER)
        greedy_bonus = jnp.where(num_match == K, bonus, PLACEHOLDER)

        out_tok = jnp.where(greedy, greedy_tok, rand_tok)
        out_bonus = jnp.where(greedy, greedy_bonus, rand_bonus)
        out_ref[...] = jnp.concatenate([out_tok, out_bonus], axis=1)


@functools.partial(jax.jit, static_argnames=())
def kernel(draft_token_ids, cu_num_draft_tokens, draft_probs, target_probs,
           bonus_token_ids, uniform_probs, q, is_greedy):
    del cu_num_draft_tokens
    t4 = target_probs.reshape(B, K, V)
    d4 = draft_probs.reshape(B, K, V)
    q3 = q.reshape(B, 1, V)
    d_tok = draft_token_ids.reshape(B, K)
    u2 = uniform_probs.reshape(B, K)
    bonus2 = bonus_token_ids.reshape(B, 1)
    greedy2 = is_greedy.astype(jnp.int32).reshape(B, 1)

    grid = (B // BB, NC)
    out = pl.pallas_call(
        _rs_kernel,
        grid=grid,
        in_specs=[
            pl.BlockSpec((BB, K), lambda i, c: (i, 0)),
            pl.BlockSpec((BB, K), lambda i, c: (i, 0)),
            pl.BlockSpec((BB, 1), lambda i, c: (i, 0)),
            pl.BlockSpec((BB, 1), lambda i, c: (i, 0)),
            pl.BlockSpec((BB, K, VB), lambda i, c: (i, 0, c)),
            pl.BlockSpec((BB, K, VB), lambda i, c: (i, 0, c)),
            pl.BlockSpec((BB, 1, VB), lambda i, c: (i, 0, c)),
        ],
        out_specs=pl.BlockSpec((BB, K + 1), lambda i, c: (i, 0)),
        out_shape=jax.ShapeDtypeStruct((B, K + 1), jnp.int32),
        scratch_shapes=[
            pltpu.VMEM((BB, K), jnp.int32),
            pltpu.VMEM((BB, K), jnp.int32),
            pltpu.VMEM((BB, K), jnp.int32),
            pltpu.VMEM((BB, K), jnp.int32),
            pltpu.VMEM((BB, K), jnp.float32),
            pltpu.VMEM((BB, K), jnp.float32),
        ],
    )(d_tok, u2, bonus2, greedy2, t4, d4, q3)
    return out


# speedup vs baseline: 2.1712x; 2.1712x over previous
import jax
import jax.numpy as jnp
from jax.experimental import pallas as pl
from jax.experimental.pallas import tpu as pltpu

B = 64
K = 4
V = 100000
PLACEHOLDER = -1

BB = 8
RB = BB * K
VB = 16384
NC = (V + VB - 1) // VB


def _scan_kernel(greedy_ref, tok_ref, t_ref, d_ref, q_ref,
                 rec_ref, targ_ref, pd_ref, pt_ref,
                 run_rv, run_ri, run_tv, run_ti, acc_pd, acc_pt):
    c = pl.program_id(1)

    @pl.when(c == 0)
    def _init():
        run_rv[...] = jnp.full((RB, 1), -1, jnp.int32)
        run_ri[...] = jnp.zeros((RB, 1), jnp.int32)
        run_tv[...] = jnp.full((RB, 1), -1, jnp.int32)
        run_ti[...] = jnp.zeros((RB, 1), jnp.int32)
        acc_pd[...] = jnp.zeros((RB, 1), jnp.float32)
        acc_pt[...] = jnp.zeros((RB, 1), jnp.float32)

    g = greedy_ref[...] != 0
    has_greedy = jnp.any(g)
    has_random = jnp.any(jnp.logical_not(g))

    t = t_ref[...]
    gid = jax.lax.broadcasted_iota(jnp.int32, (1, VB), 1) + c * VB
    valid = gid < V

    def upd_argmax(x, rv_ref, ri_ref):
        bval = jnp.max(x, axis=1, keepdims=True)
        eq = x == bval
        bidx = jnp.min(jnp.where(eq, gid, V), axis=1, keepdims=True)
        better = bval > rv_ref[...]
        ri_ref[...] = jnp.where(better, bidx, ri_ref[...])
        rv_ref[...] = jnp.maximum(rv_ref[...], bval)

    @pl.when(has_random)
    def _random_path():
        d = d_ref[...]
        qv = jnp.repeat(q_ref[...], K, axis=0)
        ratio = jnp.maximum(t - d, 0.0) / qv
        rbits = jax.lax.bitcast_convert_type(ratio, jnp.int32)
        rbits = jnp.where(rbits < 0, jnp.int32(0x7FFFFFFF), rbits)
        rkey = jnp.where(valid, rbits, -1)
        upd_argmax(rkey, run_rv, run_ri)
        tok = tok_ref[...]
        hit = gid == tok
        acc_pt[...] += jnp.sum(jnp.where(hit, t, 0.0), axis=1, keepdims=True)
        acc_pd[...] += jnp.sum(jnp.where(hit, d, 0.0), axis=1, keepdims=True)

    @pl.when(has_greedy)
    def _greedy_path():
        tkey = jnp.where(valid, jax.lax.bitcast_convert_type(t, jnp.int32), -1)
        upd_argmax(tkey, run_tv, run_ti)

    @pl.when(c == NC - 1)
    def _emit():
        rec_ref[...] = run_ri[...]
        targ_ref[...] = run_ti[...]
        pd_ref[...] = acc_pd[...]
        pt_ref[...] = acc_pt[...]


def _combine_kernel(tok_ref, u_ref, bonus_ref, greedy_ref,
                    rec_ref, targ_ref, pd_ref, pt_ref, out_ref):
    pd = pd_ref[...]
    pt = pt_ref[...]
    u = u_ref[...]
    tok = tok_ref[...]
    bonus = bonus_ref[...]
    greedy = greedy_ref[...] != 0

    r = jnp.where(pd > 0, pt / jnp.where(pd > 0, pd, 1.0), 0.0)
    accept = ((pd > 0) & (r >= u)).astype(jnp.int32)
    c1 = accept[:, 0:1]
    c2 = c1 * accept[:, 1:2]
    c3 = c2 * accept[:, 2:3]
    c4 = c3 * accept[:, 3:4]
    num_acc = c1 + c2 + c3 + c4

    pos = jax.lax.broadcasted_iota(jnp.int32, (B, K), 1)
    rand_tok = jnp.where(pos < num_acc, tok,
                         jnp.where(pos == num_acc, rec_ref[...], PLACEHOLDER))
    rand_bonus = jnp.where(num_acc == K, bonus, PLACEHOLDER)

    t_arg = targ_ref[...]
    match = (tok == t_arg).astype(jnp.int32)
    m1 = match[:, 0:1]
    m2 = m1 * match[:, 1:2]
    m3 = m2 * match[:, 2:3]
    m4 = m3 * match[:, 3:4]
    num_match = m1 + m2 + m3 + m4
    greedy_tok = jnp.where(pos <= num_match, t_arg, PLACEHOLDER)
    greedy_bonus = jnp.where(num_match == K, bonus, PLACEHOLDER)

    out_tok = jnp.where(greedy, greedy_tok, rand_tok)
    out_bonus = jnp.where(greedy, greedy_bonus, rand_bonus)
    out_ref[...] = jnp.concatenate([out_tok, out_bonus], axis=1)


def kernel(draft_token_ids, cu_num_draft_tokens, draft_probs, target_probs,
           bonus_token_ids, uniform_probs, q, is_greedy):
    del cu_num_draft_tokens
    tok_col = draft_token_ids.reshape(B * K, 1)
    greedy_col = is_greedy.astype(jnp.int32).reshape(B, 1)

    grid = (B // BB, NC)
    small = jax.ShapeDtypeStruct((B * K, 1), jnp.int32)
    smallf = jax.ShapeDtypeStruct((B * K, 1), jnp.float32)
    rec, targ, pd, pt = pl.pallas_call(
        _scan_kernel,
        grid=grid,
        in_specs=[
            pl.BlockSpec((BB, 1), lambda i, c: (i, 0)),
            pl.BlockSpec((RB, 1), lambda i, c: (i, 0)),
            pl.BlockSpec((RB, VB), lambda i, c: (i, c)),
            pl.BlockSpec((RB, VB), lambda i, c: (i, c)),
            pl.BlockSpec((BB, VB), lambda i, c: (i, c)),
        ],
        out_specs=[
            pl.BlockSpec((RB, 1), lambda i, c: (i, 0)),
            pl.BlockSpec((RB, 1), lambda i, c: (i, 0)),
            pl.BlockSpec((RB, 1), lambda i, c: (i, 0)),
            pl.BlockSpec((RB, 1), lambda i, c: (i, 0)),
        ],
        out_shape=[small, small, smallf, smallf],
        scratch_shapes=[
            pltpu.VMEM((RB, 1), jnp.int32),
            pltpu.VMEM((RB, 1), jnp.int32),
            pltpu.VMEM((RB, 1), jnp.int32),
            pltpu.VMEM((RB, 1), jnp.int32),
            pltpu.VMEM((RB, 1), jnp.float32),
            pltpu.VMEM((RB, 1), jnp.float32),
        ],
    )(greedy_col, tok_col, target_probs, draft_probs, q)

    out = pl.pallas_call(
        _combine_kernel,
        out_shape=jax.ShapeDtypeStruct((B, K + 1), jnp.int32),
    )(draft_token_ids.reshape(B, K), uniform_probs.reshape(B, K),
      bonus_token_ids.reshape(B, 1), greedy_col,
      rec.reshape(B, K), targ.reshape(B, K),
      pd.reshape(B, K), pt.reshape(B, K))
    return out


# speedup vs baseline: 2.2515x; 1.0370x over previous
import jax
import jax.numpy as jnp
from jax.experimental import pallas as pl
from jax.experimental.pallas import tpu as pltpu

B = 64
K = 4
V = 100000
PLACEHOLDER = -1

BB = 8
RB = BB * K
VB = 12544
NC = (V + VB - 1) // VB


def _scan_kernel(greedy_ref, tok_ref, t_ref, d_ref, q_ref,
                 rec_ref, targ_ref, pd_ref, pt_ref,
                 run_rv, run_ri, run_tv, run_ti, acc_pd, acc_pt):
    c = pl.program_id(1)

    @pl.when(c == 0)
    def _init():
        run_rv[...] = jnp.full((RB, 1), -1, jnp.int32)
        run_ri[...] = jnp.zeros((RB, 1), jnp.int32)
        run_tv[...] = jnp.full((RB, 1), -1, jnp.int32)
        run_ti[...] = jnp.zeros((RB, 1), jnp.int32)
        acc_pd[...] = jnp.zeros((RB, 1), jnp.float32)
        acc_pt[...] = jnp.zeros((RB, 1), jnp.float32)

    g = greedy_ref[...] != 0
    has_greedy = jnp.any(g)
    has_random = jnp.any(jnp.logical_not(g))

    t = t_ref[...]
    gid = jax.lax.broadcasted_iota(jnp.int32, (1, VB), 1) + c * VB
    valid = gid < V

    def upd_argmax(x, rv_ref, ri_ref):
        bval = jnp.max(x, axis=1, keepdims=True)
        eq = x == bval
        bidx = jnp.min(jnp.where(eq, gid, V), axis=1, keepdims=True)
        better = bval > rv_ref[...]
        ri_ref[...] = jnp.where(better, bidx, ri_ref[...])
        rv_ref[...] = jnp.maximum(rv_ref[...], bval)

    @pl.when(has_random)
    def _random_path():
        d = d_ref[...]
        qv = jnp.repeat(q_ref[...], K, axis=0)
        ratio = jnp.maximum(t - d, 0.0) / qv
        rbits = jax.lax.bitcast_convert_type(ratio, jnp.int32)
        rbits = jnp.where(rbits < 0, jnp.int32(0x7FFFFFFF), rbits)
        rkey = jnp.where(valid, rbits, -1)
        upd_argmax(rkey, run_rv, run_ri)
        tok = tok_ref[...]
        hit = gid == tok
        acc_pt[...] += jnp.sum(jnp.where(hit, t, 0.0), axis=1, keepdims=True)
        acc_pd[...] += jnp.sum(jnp.where(hit, d, 0.0), axis=1, keepdims=True)

    @pl.when(has_greedy)
    def _greedy_path():
        tkey = jnp.where(valid, jax.lax.bitcast_convert_type(t, jnp.int32), -1)
        upd_argmax(tkey, run_tv, run_ti)

    @pl.when(c == NC - 1)
    def _emit():
        rec_ref[...] = run_ri[...]
        targ_ref[...] = run_ti[...]
        pd_ref[...] = acc_pd[...]
        pt_ref[...] = acc_pt[...]


def _combine_kernel(tok_ref, u_ref, bonus_ref, greedy_ref,
                    rec_ref, targ_ref, pd_ref, pt_ref, out_ref):
    pd = pd_ref[...]
    pt = pt_ref[...]
    u = u_ref[...]
    tok = tok_ref[...]
    bonus = bonus_ref[...]
    greedy = greedy_ref[...] != 0

    r = jnp.where(pd > 0, pt / jnp.where(pd > 0, pd, 1.0), 0.0)
    accept = ((pd > 0) & (r >= u)).astype(jnp.int32)
    c1 = accept[:, 0:1]
    c2 = c1 * accept[:, 1:2]
    c3 = c2 * accept[:, 2:3]
    c4 = c3 * accept[:, 3:4]
    num_acc = c1 + c2 + c3 + c4

    pos = jax.lax.broadcasted_iota(jnp.int32, (B, K), 1)
    rand_tok = jnp.where(pos < num_acc, tok,
                         jnp.where(pos == num_acc, rec_ref[...], PLACEHOLDER))
    rand_bonus = jnp.where(num_acc == K, bonus, PLACEHOLDER)

    t_arg = targ_ref[...]
    match = (tok == t_arg).astype(jnp.int32)
    m1 = match[:, 0:1]
    m2 = m1 * match[:, 1:2]
    m3 = m2 * match[:, 2:3]
    m4 = m3 * match[:, 3:4]
    num_match = m1 + m2 + m3 + m4
    greedy_tok = jnp.where(pos <= num_match, t_arg, PLACEHOLDER)
    greedy_bonus = jnp.where(num_match == K, bonus, PLACEHOLDER)

    out_tok = jnp.where(greedy, greedy_tok, rand_tok)
    out_bonus = jnp.where(greedy, greedy_bonus, rand_bonus)
    out_ref[...] = jnp.concatenate([out_tok, out_bonus], axis=1)


def kernel(draft_token_ids, cu_num_draft_tokens, draft_probs, target_probs,
           bonus_token_ids, uniform_probs, q, is_greedy):
    del cu_num_draft_tokens
    tok_col = draft_token_ids.reshape(B * K, 1)
    greedy_col = is_greedy.astype(jnp.int32).reshape(B, 1)

    grid = (B // BB, NC)
    small = jax.ShapeDtypeStruct((B * K, 1), jnp.int32)
    smallf = jax.ShapeDtypeStruct((B * K, 1), jnp.float32)
    rec, targ, pd, pt = pl.pallas_call(
        _scan_kernel,
        grid=grid,
        in_specs=[
            pl.BlockSpec((BB, 1), lambda i, c: (i, 0)),
            pl.BlockSpec((RB, 1), lambda i, c: (i, 0)),
            pl.BlockSpec((RB, VB), lambda i, c: (i, c)),
            pl.BlockSpec((RB, VB), lambda i, c: (i, c)),
            pl.BlockSpec((BB, VB), lambda i, c: (i, c)),
        ],
        out_specs=[
            pl.BlockSpec((RB, 1), lambda i, c: (i, 0)),
            pl.BlockSpec((RB, 1), lambda i, c: (i, 0)),
            pl.BlockSpec((RB, 1), lambda i, c: (i, 0)),
            pl.BlockSpec((RB, 1), lambda i, c: (i, 0)),
        ],
        out_shape=[small, small, smallf, smallf],
        scratch_shapes=[
            pltpu.VMEM((RB, 1), jnp.int32),
            pltpu.VMEM((RB, 1), jnp.int32),
            pltpu.VMEM((RB, 1), jnp.int32),
            pltpu.VMEM((RB, 1), jnp.int32),
            pltpu.VMEM((RB, 1), jnp.float32),
            pltpu.VMEM((RB, 1), jnp.float32),
        ],
    )(greedy_col, tok_col, target_probs, draft_probs, q)

    out = pl.pallas_call(
        _combine_kernel,
        out_shape=jax.ShapeDtypeStruct((B, K + 1), jnp.int32),
    )(draft_token_ids.reshape(B, K), uniform_probs.reshape(B, K),
      bonus_token_ids.reshape(B, 1), greedy_col,
      rec.reshape(B, K), targ.reshape(B, K),
      pd.reshape(B, K), pt.reshape(B, K))
    return out


# speedup vs baseline: 2.2704x; 1.0084x over previous
import jax
import jax.numpy as jnp
from jax.experimental import pallas as pl
from jax.experimental.pallas import tpu as pltpu

B = 64
K = 4
V = 100000
PLACEHOLDER = -1

BB = 8
RB = BB * K
VB = 12544
NC = (V + VB - 1) // VB


def _scan_kernel(allg_ref, greedy_ref, tok_ref, t_ref, d_ref, q_ref,
                 rec_ref, targ_ref, pd_ref, pt_ref,
                 run_rv, run_ri, run_tv, run_ti, acc_pd, acc_pt):
    del allg_ref
    c = pl.program_id(1)

    @pl.when(c == 0)
    def _init():
        run_rv[...] = jnp.full((RB, 1), -1, jnp.int32)
        run_ri[...] = jnp.zeros((RB, 1), jnp.int32)
        run_tv[...] = jnp.full((RB, 1), -1, jnp.int32)
        run_ti[...] = jnp.zeros((RB, 1), jnp.int32)
        acc_pd[...] = jnp.zeros((RB, 1), jnp.float32)
        acc_pt[...] = jnp.zeros((RB, 1), jnp.float32)

    g = greedy_ref[...] != 0
    has_greedy = jnp.any(g)
    has_random = jnp.any(jnp.logical_not(g))

    t = t_ref[...]
    gid = jax.lax.broadcasted_iota(jnp.int32, (1, VB), 1) + c * VB
    valid = gid < V

    def upd_argmax(x, rv_ref, ri_ref):
        bval = jnp.max(x, axis=1, keepdims=True)
        eq = x == bval
        bidx = jnp.min(jnp.where(eq, gid, V), axis=1, keepdims=True)
        better = bval > rv_ref[...]
        ri_ref[...] = jnp.where(better, bidx, ri_ref[...])
        rv_ref[...] = jnp.maximum(rv_ref[...], bval)

    @pl.when(has_random)
    def _random_path():
        d = d_ref[...]
        qv = jnp.repeat(q_ref[...], K, axis=0)
        ratio = jnp.maximum(t - d, 0.0) / qv
        rbits = jax.lax.bitcast_convert_type(ratio, jnp.int32)
        rbits = jnp.where(rbits < 0, jnp.int32(0x7FFFFFFF), rbits)
        rkey = jnp.where(valid, rbits, -1)
        upd_argmax(rkey, run_rv, run_ri)
        tok = tok_ref[...]
        hit = gid == tok
        acc_pt[...] += jnp.sum(jnp.where(hit, t, 0.0), axis=1, keepdims=True)
        acc_pd[...] += jnp.sum(jnp.where(hit, d, 0.0), axis=1, keepdims=True)

    @pl.when(has_greedy)
    def _greedy_path():
        tkey = jnp.where(valid, jax.lax.bitcast_convert_type(t, jnp.int32), -1)
        upd_argmax(tkey, run_tv, run_ti)

    @pl.when(c == NC - 1)
    def _emit():
        rec_ref[...] = run_ri[...]
        targ_ref[...] = run_ti[...]
        pd_ref[...] = acc_pd[...]
        pt_ref[...] = acc_pt[...]


def _combine_kernel(tok_ref, u_ref, bonus_ref, greedy_ref,
                    rec_ref, targ_ref, pd_ref, pt_ref, out_ref):
    pd = pd_ref[...]
    pt = pt_ref[...]
    u = u_ref[...]
    tok = tok_ref[...]
    bonus = bonus_ref[...]
    greedy = greedy_ref[...] != 0

    r = jnp.where(pd > 0, pt / jnp.where(pd > 0, pd, 1.0), 0.0)
    accept = ((pd > 0) & (r >= u)).astype(jnp.int32)
    c1 = accept[:, 0:1]
    c2 = c1 * accept[:, 1:2]
    c3 = c2 * accept[:, 2:3]
    c4 = c3 * accept[:, 3:4]
    num_acc = c1 + c2 + c3 + c4

    pos = jax.lax.broadcasted_iota(jnp.int32, (B, K), 1)
    rand_tok = jnp.where(pos < num_acc, tok,
                         jnp.where(pos == num_acc, rec_ref[...], PLACEHOLDER))
    rand_bonus = jnp.where(num_acc == K, bonus, PLACEHOLDER)

    t_arg = targ_ref[...]
    match = (tok == t_arg).astype(jnp.int32)
    m1 = match[:, 0:1]
    m2 = m1 * match[:, 1:2]
    m3 = m2 * match[:, 2:3]
    m4 = m3 * match[:, 3:4]
    num_match = m1 + m2 + m3 + m4
    greedy_tok = jnp.where(pos <= num_match, t_arg, PLACEHOLDER)
    greedy_bonus = jnp.where(num_match == K, bonus, PLACEHOLDER)

    out_tok = jnp.where(greedy, greedy_tok, rand_tok)
    out_bonus = jnp.where(greedy, greedy_bonus, rand_bonus)
    out_ref[...] = jnp.concatenate([out_tok, out_bonus], axis=1)


def kernel(draft_token_ids, cu_num_draft_tokens, draft_probs, target_probs,
           bonus_token_ids, uniform_probs, q, is_greedy):
    del cu_num_draft_tokens
    tok_col = draft_token_ids.reshape(B * K, 1)
    greedy_col = is_greedy.astype(jnp.int32).reshape(B, 1)
    all_greedy = jnp.all(is_greedy.reshape(B // BB, BB), axis=1).astype(jnp.int32)

    def _dq_map(i, c, s):
        skip = s[i] == 1
        return jnp.where(skip, 0, i), jnp.where(skip, 0, c)

    small = jax.ShapeDtypeStruct((B * K, 1), jnp.int32)
    smallf = jax.ShapeDtypeStruct((B * K, 1), jnp.float32)
    rec, targ, pd, pt = pl.pallas_call(
        _scan_kernel,
        grid_spec=pltpu.PrefetchScalarGridSpec(
            num_scalar_prefetch=1,
            grid=(B // BB, NC),
            in_specs=[
                pl.BlockSpec((BB, 1), lambda i, c, s: (i, 0)),
                pl.BlockSpec((RB, 1), lambda i, c, s: (i, 0)),
                pl.BlockSpec((RB, VB), lambda i, c, s: (i, c)),
                pl.BlockSpec((RB, VB), _dq_map),
                pl.BlockSpec((BB, VB), _dq_map),
            ],
            out_specs=[
                pl.BlockSpec((RB, 1), lambda i, c, s: (i, 0)),
                pl.BlockSpec((RB, 1), lambda i, c, s: (i, 0)),
                pl.BlockSpec((RB, 1), lambda i, c, s: (i, 0)),
                pl.BlockSpec((RB, 1), lambda i, c, s: (i, 0)),
            ],
            scratch_shapes=[
                pltpu.VMEM((RB, 1), jnp.int32),
                pltpu.VMEM((RB, 1), jnp.int32),
                pltpu.VMEM((RB, 1), jnp.int32),
                pltpu.VMEM((RB, 1), jnp.int32),
                pltpu.VMEM((RB, 1), jnp.float32),
                pltpu.VMEM((RB, 1), jnp.float32),
            ],
        ),
        out_shape=[small, small, smallf, smallf],
    )(all_greedy, greedy_col, tok_col, target_probs, draft_probs, q)

    out = pl.pallas_call(
        _combine_kernel,
        out_shape=jax.ShapeDtypeStruct((B, K + 1), jnp.int32),
    )(draft_token_ids.reshape(B, K), uniform_probs.reshape(B, K),
      bonus_token_ids.reshape(B, 1), greedy_col,
      rec.reshape(B, K), targ.reshape(B, K),
      pd.reshape(B, K), pt.reshape(B, K))
    return out


# speedup vs baseline: 2.3272x; 1.0250x over previous
import jax
import jax.numpy as jnp
from jax.experimental import pallas as pl
from jax.experimental.pallas import tpu as pltpu

B = 64
K = 4
V = 100000
PLACEHOLDER = -1

BB = 8
RB = BB * K
VB = 25088
NC = (V + VB - 1) // VB


def _scan_kernel(allg_ref, greedy_ref, tok_ref, t_ref, d_ref, q_ref,
                 rec_ref, targ_ref, pd_ref, pt_ref,
                 run_rv, run_ri, run_tv, run_ti, acc_pd, acc_pt):
    del allg_ref
    c = pl.program_id(1)

    @pl.when(c == 0)
    def _init():
        run_rv[...] = jnp.full((RB, 1), -1, jnp.int32)
        run_ri[...] = jnp.zeros((RB, 1), jnp.int32)
        run_tv[...] = jnp.full((RB, 1), -1, jnp.int32)
        run_ti[...] = jnp.zeros((RB, 1), jnp.int32)
        acc_pd[...] = jnp.zeros((RB, 1), jnp.float32)
        acc_pt[...] = jnp.zeros((RB, 1), jnp.float32)

    g = greedy_ref[...] != 0
    has_greedy = jnp.any(g)
    has_random = jnp.any(jnp.logical_not(g))

    t = t_ref[...]
    gid = jax.lax.broadcasted_iota(jnp.int32, (1, VB), 1) + c * VB
    valid = gid < V

    def upd_argmax(x, rv_ref, ri_ref):
        bval = jnp.max(x, axis=1, keepdims=True)
        eq = x == bval
        bidx = jnp.min(jnp.where(eq, gid, V), axis=1, keepdims=True)
        better = bval > rv_ref[...]
        ri_ref[...] = jnp.where(better, bidx, ri_ref[...])
        rv_ref[...] = jnp.maximum(rv_ref[...], bval)

    @pl.when(has_random)
    def _random_path():
        d = d_ref[...]
        qv = jnp.repeat(q_ref[...], K, axis=0)
        ratio = jnp.maximum(t - d, 0.0) / qv
        rbits = jax.lax.bitcast_convert_type(ratio, jnp.int32)
        rbits = jnp.where(rbits < 0, jnp.int32(0x7FFFFFFF), rbits)
        rkey = jnp.where(valid, rbits, -1)
        upd_argmax(rkey, run_rv, run_ri)
        tok = tok_ref[...]
        hit = gid == tok
        acc_pt[...] += jnp.sum(jnp.where(hit, t, 0.0), axis=1, keepdims=True)
        acc_pd[...] += jnp.sum(jnp.where(hit, d, 0.0), axis=1, keepdims=True)

    @pl.when(has_greedy)
    def _greedy_path():
        tkey = jnp.where(valid, jax.lax.bitcast_convert_type(t, jnp.int32), -1)
        upd_argmax(tkey, run_tv, run_ti)

    @pl.when(c == NC - 1)
    def _emit():
        rec_ref[...] = run_ri[...]
        targ_ref[...] = run_ti[...]
        pd_ref[...] = acc_pd[...]
        pt_ref[...] = acc_pt[...]


def _combine_kernel(tok_ref, u_ref, bonus_ref, greedy_ref,
                    rec_ref, targ_ref, pd_ref, pt_ref, out_ref):
    pd = pd_ref[...]
    pt = pt_ref[...]
    u = u_ref[...]
    tok = tok_ref[...]
    bonus = bonus_ref[...]
    greedy = greedy_ref[...] != 0

    r = jnp.where(pd > 0, pt / jnp.where(pd > 0, pd, 1.0), 0.0)
    accept = ((pd > 0) & (r >= u)).astype(jnp.int32)
    c1 = accept[:, 0:1]
    c2 = c1 * accept[:, 1:2]
    c3 = c2 * accept[:, 2:3]
    c4 = c3 * accept[:, 3:4]
    num_acc = c1 + c2 + c3 + c4

    pos = jax.lax.broadcasted_iota(jnp.int32, (B, K), 1)
    rand_tok = jnp.where(pos < num_acc, tok,
                         jnp.where(pos == num_acc, rec_ref[...], PLACEHOLDER))
    rand_bonus = jnp.where(num_acc == K, bonus, PLACEHOLDER)

    t_arg = targ_ref[...]
    match = (tok == t_arg).astype(jnp.int32)
    m1 = match[:, 0:1]
    m2 = m1 * match[:, 1:2]
    m3 = m2 * match[:, 2:3]
    m4 = m3 * match[:, 3:4]
    num_match = m1 + m2 + m3 + m4
    greedy_tok = jnp.where(pos <= num_match, t_arg, PLACEHOLDER)
    greedy_bonus = jnp.where(num_match == K, bonus, PLACEHOLDER)

    out_tok = jnp.where(greedy, greedy_tok, rand_tok)
    out_bonus = jnp.where(greedy, greedy_bonus, rand_bonus)
    out_ref[...] = jnp.concatenate([out_tok, out_bonus], axis=1)


def kernel(draft_token_ids, cu_num_draft_tokens, draft_probs, target_probs,
           bonus_token_ids, uniform_probs, q, is_greedy):
    del cu_num_draft_tokens
    tok_col = draft_token_ids.reshape(B * K, 1)
    greedy_col = is_greedy.astype(jnp.int32).reshape(B, 1)
    all_greedy = jnp.all(is_greedy.reshape(B // BB, BB), axis=1).astype(jnp.int32)

    def _dq_map(i, c, s):
        skip = s[i] == 1
        return jnp.where(skip, 0, i), jnp.where(skip, 0, c)

    small = jax.ShapeDtypeStruct((B * K, 1), jnp.int32)
    smallf = jax.ShapeDtypeStruct((B * K, 1), jnp.float32)
    rec, targ, pd, pt = pl.pallas_call(
        _scan_kernel,
        grid_spec=pltpu.PrefetchScalarGridSpec(
            num_scalar_prefetch=1,
            grid=(B // BB, NC),
            in_specs=[
                pl.BlockSpec((BB, 1), lambda i, c, s: (i, 0)),
                pl.BlockSpec((RB, 1), lambda i, c, s: (i, 0)),
                pl.BlockSpec((RB, VB), lambda i, c, s: (i, c)),
                pl.BlockSpec((RB, VB), _dq_map),
                pl.BlockSpec((BB, VB), _dq_map),
            ],
            out_specs=[
                pl.BlockSpec((RB, 1), lambda i, c, s: (i, 0)),
                pl.BlockSpec((RB, 1), lambda i, c, s: (i, 0)),
                pl.BlockSpec((RB, 1), lambda i, c, s: (i, 0)),
                pl.BlockSpec((RB, 1), lambda i, c, s: (i, 0)),
            ],
            scratch_shapes=[
                pltpu.VMEM((RB, 1), jnp.int32),
                pltpu.VMEM((RB, 1), jnp.int32),
                pltpu.VMEM((RB, 1), jnp.int32),
                pltpu.VMEM((RB, 1), jnp.int32),
                pltpu.VMEM((RB, 1), jnp.float32),
                pltpu.VMEM((RB, 1), jnp.float32),
            ],
        ),
        out_shape=[small, small, smallf, smallf],
    )(all_greedy, greedy_col, tok_col, target_probs, draft_probs, q)

    out = pl.pallas_call(
        _combine_kernel,
        out_shape=jax.ShapeDtypeStruct((B, K + 1), jnp.int32),
    )(draft_token_ids.reshape(B, K), uniform_probs.reshape(B, K),
      bonus_token_ids.reshape(B, 1), greedy_col,
      rec.reshape(B, K), targ.reshape(B, K),
      pd.reshape(B, K), pt.reshape(B, K))
    return out


# speedup vs baseline: 2.3619x; 1.0149x over previous
import jax
import jax.numpy as jnp
from jax.experimental import pallas as pl
from jax.experimental.pallas import tpu as pltpu

B = 64
K = 4
V = 100000
PLACEHOLDER = -1

BB = 16
RB = BB * K
VB = 25088
NC = (V + VB - 1) // VB


def _scan_kernel(allg_ref, greedy_ref, tok_ref, t_ref, d_ref, q_ref,
                 rec_ref, targ_ref, pd_ref, pt_ref,
                 run_rv, run_ri, run_tv, run_ti, acc_pd, acc_pt):
    del allg_ref
    c = pl.program_id(1)

    @pl.when(c == 0)
    def _init():
        run_rv[...] = jnp.full((RB, 1), -1, jnp.int32)
        run_ri[...] = jnp.zeros((RB, 1), jnp.int32)
        run_tv[...] = jnp.full((RB, 1), -1, jnp.int32)
        run_ti[...] = jnp.zeros((RB, 1), jnp.int32)
        acc_pd[...] = jnp.zeros((RB, 1), jnp.float32)
        acc_pt[...] = jnp.zeros((RB, 1), jnp.float32)

    g = greedy_ref[...] != 0
    has_greedy = jnp.any(g)
    has_random = jnp.any(jnp.logical_not(g))

    t = t_ref[...]
    gid = jax.lax.broadcasted_iota(jnp.int32, (1, VB), 1) + c * VB
    valid = gid < V

    def upd_argmax(x, rv_ref, ri_ref):
        bval = jnp.max(x, axis=1, keepdims=True)
        eq = x == bval
        bidx = jnp.min(jnp.where(eq, gid, V), axis=1, keepdims=True)
        better = bval > rv_ref[...]
        ri_ref[...] = jnp.where(better, bidx, ri_ref[...])
        rv_ref[...] = jnp.maximum(rv_ref[...], bval)

    @pl.when(has_random)
    def _random_path():
        d = d_ref[...]
        qv = jnp.repeat(q_ref[...], K, axis=0)
        ratio = jnp.maximum(t - d, 0.0) / qv
        rbits = jax.lax.bitcast_convert_type(ratio, jnp.int32)
        rbits = jnp.where(rbits < 0, jnp.int32(0x7FFFFFFF), rbits)
        rkey = jnp.where(valid, rbits, -1)
        upd_argmax(rkey, run_rv, run_ri)
        tok = tok_ref[...]
        hit = gid == tok
        acc_pt[...] += jnp.sum(jnp.where(hit, t, 0.0), axis=1, keepdims=True)
        acc_pd[...] += jnp.sum(jnp.where(hit, d, 0.0), axis=1, keepdims=True)

    @pl.when(has_greedy)
    def _greedy_path():
        tkey = jnp.where(valid, jax.lax.bitcast_convert_type(t, jnp.int32), -1)
        upd_argmax(tkey, run_tv, run_ti)

    @pl.when(c == NC - 1)
    def _emit():
        rec_ref[...] = run_ri[...]
        targ_ref[...] = run_ti[...]
        pd_ref[...] = acc_pd[...]
        pt_ref[...] = acc_pt[...]


def _combine_kernel(tok_ref, u_ref, bonus_ref, greedy_ref,
                    rec_ref, targ_ref, pd_ref, pt_ref, out_ref):
    pd = pd_ref[...]
    pt = pt_ref[...]
    u = u_ref[...]
    tok = tok_ref[...]
    bonus = bonus_ref[...]
    greedy = greedy_ref[...] != 0

    r = jnp.where(pd > 0, pt / jnp.where(pd > 0, pd, 1.0), 0.0)
    accept = ((pd > 0) & (r >= u)).astype(jnp.int32)
    c1 = accept[:, 0:1]
    c2 = c1 * accept[:, 1:2]
    c3 = c2 * accept[:, 2:3]
    c4 = c3 * accept[:, 3:4]
    num_acc = c1 + c2 + c3 + c4

    pos = jax.lax.broadcasted_iota(jnp.int32, (B, K), 1)
    rand_tok = jnp.where(pos < num_acc, tok,
                         jnp.where(pos == num_acc, rec_ref[...], PLACEHOLDER))
    rand_bonus = jnp.where(num_acc == K, bonus, PLACEHOLDER)

    t_arg = targ_ref[...]
    match = (tok == t_arg).astype(jnp.int32)
    m1 = match[:, 0:1]
    m2 = m1 * match[:, 1:2]
    m3 = m2 * match[:, 2:3]
    m4 = m3 * match[:, 3:4]
    num_match = m1 + m2 + m3 + m4
    greedy_tok = jnp.where(pos <= num_match, t_arg, PLACEHOLDER)
    greedy_bonus = jnp.where(num_match == K, bonus, PLACEHOLDER)

    out_tok = jnp.where(greedy, greedy_tok, rand_tok)
    out_bonus = jnp.where(greedy, greedy_bonus, rand_bonus)
    out_ref[...] = jnp.concatenate([out_tok, out_bonus], axis=1)


def kernel(draft_token_ids, cu_num_draft_tokens, draft_probs, target_probs,
           bonus_token_ids, uniform_probs, q, is_greedy):
    del cu_num_draft_tokens
    tok_col = draft_token_ids.reshape(B * K, 1)
    greedy_col = is_greedy.astype(jnp.int32).reshape(B, 1)
    all_greedy = jnp.all(is_greedy.reshape(B // BB, BB), axis=1).astype(jnp.int32)

    def _dq_map(i, c, s):
        skip = s[i] == 1
        return jnp.where(skip, 0, i), jnp.where(skip, 0, c)

    small = jax.ShapeDtypeStruct((B * K, 1), jnp.int32)
    smallf = jax.ShapeDtypeStruct((B * K, 1), jnp.float32)
    rec, targ, pd, pt = pl.pallas_call(
        _scan_kernel,
        grid_spec=pltpu.PrefetchScalarGridSpec(
            num_scalar_prefetch=1,
            grid=(B // BB, NC),
            in_specs=[
                pl.BlockSpec((BB, 1), lambda i, c, s: (i, 0)),
                pl.BlockSpec((RB, 1), lambda i, c, s: (i, 0)),
                pl.BlockSpec((RB, VB), lambda i, c, s: (i, c)),
                pl.BlockSpec((RB, VB), _dq_map),
                pl.BlockSpec((BB, VB), _dq_map),
            ],
            out_specs=[
                pl.BlockSpec((RB, 1), lambda i, c, s: (i, 0)),
                pl.BlockSpec((RB, 1), lambda i, c, s: (i, 0)),
                pl.BlockSpec((RB, 1), lambda i, c, s: (i, 0)),
                pl.BlockSpec((RB, 1), lambda i, c, s: (i, 0)),
            ],
            scratch_shapes=[
                pltpu.VMEM((RB, 1), jnp.int32),
                pltpu.VMEM((RB, 1), jnp.int32),
                pltpu.VMEM((RB, 1), jnp.int32),
                pltpu.VMEM((RB, 1), jnp.int32),
                pltpu.VMEM((RB, 1), jnp.float32),
                pltpu.VMEM((RB, 1), jnp.float32),
            ],
        ),
        out_shape=[small, small, smallf, smallf],
    )(all_greedy, greedy_col, tok_col, target_probs, draft_probs, q)

    out = pl.pallas_call(
        _combine_kernel,
        out_shape=jax.ShapeDtypeStruct((B, K + 1), jnp.int32),
    )(draft_token_ids.reshape(B, K), uniform_probs.reshape(B, K),
      bonus_token_ids.reshape(B, 1), greedy_col,
      rec.reshape(B, K), targ.reshape(B, K),
      pd.reshape(B, K), pt.reshape(B, K))
    return out
